# baseline (speedup 1.0000x reference)
import functools

import jax
import jax.numpy as jnp
from jax import lax
from jax.experimental import pallas as pl
from jax.experimental.pallas import tpu as pltpu
from jax.experimental.pallas import tpu_sc as plsc


def _sc_gather(table, idx_flat, d_model):
    try:
        info = plsc.get_sparse_core_info()
        nc, ns = info.num_cores, info.num_subcores
    except Exception:
        nc, ns = 2, 16
    nw = nc * ns
    b = idx_flat.shape[0]
    b_per_w = b // nw

    mesh = plsc.VectorSubcoreMesh(core_axis_name="c", subcore_axis_name="s")

    @functools.partial(
        pl.kernel,
        mesh=mesh,
        compiler_params=pltpu.CompilerParams(use_tc_tiling_on_sc=False),
        out_type=jax.ShapeDtypeStruct((b, d_model), jnp.float32),
        scratch_types=[
            pltpu.VMEM((b_per_w,), jnp.int32),
            pltpu.VMEM((b_per_w, d_model), jnp.float32),
            pltpu.SemaphoreType.DMA,
        ],
    )
    def gather_kernel(table_hbm, idx_hbm, out_hbm, idx_v, rows_v, sem):
        wid = lax.axis_index("s") * nc + lax.axis_index("c")
        base = wid * b_per_w
        pltpu.sync_copy(idx_hbm.at[pl.ds(base, b_per_w)], idx_v)
        pltpu.async_copy(table_hbm.at[idx_v], rows_v, sem).wait()
        pltpu.sync_copy(rows_v, out_hbm.at[pl.ds(base, b_per_w)])

    return gather_kernel(table, idx_flat)


def _tc_project(h_flat, fc_w, fc_b2d, block_m):
    m, d_model = h_flat.shape
    vocab = fc_w.shape[0]

    def mm_kernel(h_ref, w_ref, b_ref, o_ref):
        acc = lax.dot_general(
            h_ref[...],
            w_ref[...],
            (((1,), (1,)), ((), ())),
            preferred_element_type=jnp.float32,
        )
        o_ref[...] = acc + b_ref[...]

    return pl.pallas_call(
        mm_kernel,
        grid=(m // block_m,),
        in_specs=[
            pl.BlockSpec((block_m, d_model), lambda i: (i, 0)),
            pl.BlockSpec((vocab, d_model), lambda i: (0, 0)),
            pl.BlockSpec((1, vocab), lambda i: (0, 0)),
        ],
        out_specs=pl.BlockSpec((block_m, vocab), lambda i: (i, 0)),
        out_shape=jax.ShapeDtypeStruct((m, vocab), jnp.float32),
    )(h_flat, fc_w, fc_b2d)


def kernel(x, emb_table, fc_w, fc_b):
    batch, seq = x.shape
    vocab, d_model = emb_table.shape
    idx_flat = x.reshape(-1).astype(jnp.int32)
    h_flat = _sc_gather(emb_table, idx_flat, d_model)
    out_flat = _tc_project(h_flat, fc_w, fc_b.reshape(1, vocab), block_m=2048)
    return out_flat.reshape(batch, seq, vocab)
